# baseline (speedup 1.0000x reference)
import functools

import jax
import jax.numpy as jnp
from jax import lax
from jax.experimental import pallas as pl
from jax.experimental.pallas import tpu as pltpu
from jax.experimental.pallas import tpu_sc as plsc

BATCH = 1024
SEQ = 128
VOCAB = 100000
D = 128

NC = 2
NS = 16
NW = NC * NS
TOK = BATCH * SEQ
TPW = TOK // NW
G = 128
KCH = TPW // G
NBUF = 4


def _sc_body(ids_hbm, mask_hbm, table_hbm, out_hbm, idx_v, mask_v,
             rows, gs, os):
    c = lax.axis_index("c")
    s = lax.axis_index("s")
    wid = s * NC + c
    base = wid * TPW

    pltpu.sync_copy(ids_hbm.at[wid], idx_v)
    pltpu.sync_copy(mask_hbm.at[wid], mask_v)

    ghandle = [None] * NBUF
    ohandle = [None] * NBUF

    def multiply(rv, g):
        def tok(t, carry2):
            midx = jnp.full((16,), 0, jnp.int32) + (g * G + t)
            m = plsc.load_gather(mask_v, [midx]).astype(jnp.float32)
            for j in range(D // 16):
                sl = pl.ds(j * 16, 16)
                rv[t, sl] = rv[t, sl] * m
            return carry2

        lax.fori_loop(0, G, tok, 0)

    def start_gather(g):
        b = g % NBUF
        ghandle[b] = pltpu.async_copy(
            table_hbm.at[idx_v.at[g]], rows[b], gs[b])

    for g in range(NBUF - 1):
        start_gather(g)

    for g in range(KCH):
        b = g % NBUF
        gn = g + NBUF - 1
        if gn < KCH:
            bn = gn % NBUF
            if ohandle[bn] is not None:
                ohandle[bn].wait()
                ohandle[bn] = None
            start_gather(gn)
        ghandle[b].wait()
        multiply(rows[b], g)
        ohandle[b] = pltpu.async_copy(
            rows[b], out_hbm.at[pl.ds(base + g * G, G)], os[b])
    for b in range(NBUF):
        if ohandle[b] is not None:
            ohandle[b].wait()


def _sc_entry(ids_hbm, mask_hbm, table_hbm, out_hbm, *scratch):
    rows = scratch[2:2 + NBUF]
    gs = scratch[2 + NBUF:2 + 2 * NBUF]
    os = scratch[2 + 2 * NBUF:2 + 3 * NBUF]
    _sc_body(ids_hbm, mask_hbm, table_hbm, out_hbm, scratch[0], scratch[1],
             rows, gs, os)


@jax.jit
def _sc_call(ids, mask, table):
    mesh = plsc.VectorSubcoreMesh(core_axis_name="c", subcore_axis_name="s")
    kfn = functools.partial(
        pl.kernel,
        mesh=mesh,
        out_type=jax.ShapeDtypeStruct((TOK, D), jnp.float32),
        scratch_types=[
            pltpu.VMEM((KCH, G), jnp.int32),
            pltpu.VMEM((TPW,), jnp.int32),
        ] + [pltpu.VMEM((G, D), jnp.float32) for _ in range(NBUF)]
          + [pltpu.SemaphoreType.DMA for _ in range(2 * NBUF)],
        compiler_params=pltpu.CompilerParams(needs_layout_passes=False),
    )(_sc_entry)
    return kfn(ids, mask, table)


def kernel(input_ids, attention_mask, embedding_table):
    ids = input_ids.reshape(NW, KCH, G)
    mask = attention_mask.reshape(NW, TPW)
    out = _sc_call(ids, mask, embedding_table)
    return out.reshape(BATCH, SEQ, D)


# speedup vs baseline: 1.2794x; 1.2794x over previous
import functools

import jax
import jax.numpy as jnp
from jax import lax
from jax.experimental import pallas as pl
from jax.experimental.pallas import tpu as pltpu
from jax.experimental.pallas import tpu_sc as plsc

BATCH = 1024
SEQ = 128
VOCAB = 100000
D = 128

NC = 2
NS = 16
NW = NC * NS
TOK = BATCH * SEQ
TPW = TOK // NW
G = 128
KCH = TPW // G
NBUF = 6
GA = 2


def _sc_body(ids_hbm, mask_hbm, table_hbm, out_hbm, idx_v, mask_v,
             rows, gs, os):
    c = lax.axis_index("c")
    s = lax.axis_index("s")
    wid = s * NC + c
    base = wid * TPW

    pltpu.sync_copy(ids_hbm.at[wid], idx_v)
    pltpu.sync_copy(mask_hbm.at[wid], mask_v)

    ghandle = [None] * NBUF
    ohandle = [None] * NBUF

    def multiply(rv, g):
        def tok(t, carry2):
            midx = jnp.full((16,), 0, jnp.int32) + (g * G + t)
            m = plsc.load_gather(mask_v, [midx])
            for j in range(D // 16):
                sl = pl.ds(j * 16, 16)
                rv[t, sl] = rv[t, sl] * m
            return carry2

        lax.fori_loop(0, G, tok, 0)

    def start_gather(g):
        b = g % NBUF
        ghandle[b] = pltpu.async_copy(
            table_hbm.at[idx_v.at[g]], rows[b], gs[b])

    for g in range(GA + 1):
        start_gather(g)

    for g in range(KCH):
        b = g % NBUF
        gn = g + GA + 1
        if gn < KCH:
            bn = gn % NBUF
            if ohandle[bn] is not None:
                ohandle[bn].wait()
                ohandle[bn] = None
            start_gather(gn)
        ghandle[b].wait()
        multiply(rows[b], g)
        ohandle[b] = pltpu.async_copy(
            rows[b], out_hbm.at[pl.ds(base + g * G, G)], os[b])
    for b in range(NBUF):
        if ohandle[b] is not None:
            ohandle[b].wait()


def _sc_entry(ids_hbm, mask_hbm, table_hbm, out_hbm, *scratch):
    rows = scratch[2:2 + NBUF]
    gs = scratch[2 + NBUF:2 + 2 * NBUF]
    os = scratch[2 + 2 * NBUF:2 + 3 * NBUF]
    _sc_body(ids_hbm, mask_hbm, table_hbm, out_hbm, scratch[0], scratch[1],
             rows, gs, os)


@jax.jit
def _sc_call(ids, mask, table):
    mesh = plsc.VectorSubcoreMesh(core_axis_name="c", subcore_axis_name="s")
    kfn = functools.partial(
        pl.kernel,
        mesh=mesh,
        out_type=jax.ShapeDtypeStruct((TOK, D), jnp.float32),
        scratch_types=[
            pltpu.VMEM((KCH, G), jnp.int32),
            pltpu.VMEM((TPW,), jnp.float32),
        ] + [pltpu.VMEM((G, D), jnp.float32) for _ in range(NBUF)]
          + [pltpu.SemaphoreType.DMA for _ in range(2 * NBUF)],
        compiler_params=pltpu.CompilerParams(needs_layout_passes=False),
    )(_sc_entry)
    return kfn(ids, mask, table)


def kernel(input_ids, attention_mask, embedding_table):
    ids = input_ids.reshape(NW, KCH, G)
    mask = attention_mask.reshape(NW, TPW).astype(jnp.float32)
    out = _sc_call(ids, mask, embedding_table)
    return out.reshape(BATCH, SEQ, D)


# speedup vs baseline: 1.2795x; 1.0001x over previous
import functools

import jax
import jax.numpy as jnp
from jax import lax
from jax.experimental import pallas as pl
from jax.experimental.pallas import tpu as pltpu
from jax.experimental.pallas import tpu_sc as plsc

BATCH = 1024
SEQ = 128
VOCAB = 100000
D = 128

NC = 2
NS = 16
NW = NC * NS
TOK = BATCH * SEQ
TPW = TOK // NW
G = 128
KCH = TPW // G
NBUF = 7
GA = 3


def _sc_body(ids_hbm, mask_hbm, table_hbm, out_hbm, idx_v, mask_iv, mask_v,
             rows, gs, os):
    c = lax.axis_index("c")
    s = lax.axis_index("s")
    wid = s * NC + c
    base = wid * TPW

    pltpu.sync_copy(ids_hbm.at[pl.ds(wid * KCH, KCH)], idx_v)
    pltpu.sync_copy(mask_hbm.at[pl.ds(wid * KCH, KCH)], mask_iv)

    def cvt_row(r, carry):
        for cgrp in range(G // 16):
            sl = pl.ds(cgrp * 16, 16)
            mask_v[pl.ds(r * G + cgrp * 16, 16)] = mask_iv[r, sl].astype(
                jnp.float32)
        return carry

    lax.fori_loop(0, KCH, cvt_row, 0)

    ghandle = [None] * NBUF
    ohandle = [None] * NBUF

    def multiply(rv, g):
        def tok(t, carry2):
            midx = jnp.full((16,), 0, jnp.int32) + (g * G + t)
            m = plsc.load_gather(mask_v, [midx])
            for j in range(D // 16):
                sl = pl.ds(j * 16, 16)
                rv[t, sl] = rv[t, sl] * m
            return carry2

        lax.fori_loop(0, G, tok, 0)

    def start_gather(g):
        b = g % NBUF
        ghandle[b] = pltpu.async_copy(
            table_hbm.at[idx_v.at[g]], rows[b], gs[b])

    for g in range(GA + 1):
        start_gather(g)

    for g in range(KCH):
        b = g % NBUF
        gn = g + GA + 1
        if gn < KCH:
            bn = gn % NBUF
            if ohandle[bn] is not None:
                ohandle[bn].wait()
                ohandle[bn] = None
            start_gather(gn)
        ghandle[b].wait()
        multiply(rows[b], g)
        ohandle[b] = pltpu.async_copy(
            rows[b], out_hbm.at[pl.ds(base + g * G, G)], os[b])
    for b in range(NBUF):
        if ohandle[b] is not None:
            ohandle[b].wait()


def _sc_entry(ids_hbm, mask_hbm, table_hbm, out_hbm, *scratch):
    rows = scratch[3:3 + NBUF]
    gs = scratch[3 + NBUF:3 + 2 * NBUF]
    os = scratch[3 + 2 * NBUF:3 + 3 * NBUF]
    _sc_body(ids_hbm, mask_hbm, table_hbm, out_hbm, scratch[0], scratch[1],
             scratch[2], rows, gs, os)


@jax.jit
def _sc_call(ids, mask, table):
    mesh = plsc.VectorSubcoreMesh(core_axis_name="c", subcore_axis_name="s")
    kfn = functools.partial(
        pl.kernel,
        mesh=mesh,
        out_type=jax.ShapeDtypeStruct((TOK, D), jnp.float32),
        scratch_types=[
            pltpu.VMEM((KCH, G), jnp.int32),
            pltpu.VMEM((KCH, G), jnp.int32),
            pltpu.VMEM((TPW,), jnp.float32),
        ] + [pltpu.VMEM((G, D), jnp.float32) for _ in range(NBUF)]
          + [pltpu.SemaphoreType.DMA for _ in range(2 * NBUF)],
        compiler_params=pltpu.CompilerParams(needs_layout_passes=False),
    )(_sc_entry)
    return kfn(ids, mask, table)


def kernel(input_ids, attention_mask, embedding_table):
    out = _sc_call(input_ids, attention_mask, embedding_table)
    return out.reshape(BATCH, SEQ, D)
